# baseline (speedup 1.0000x reference)
import functools

import jax
import jax.numpy as jnp
from jax import lax
from jax.experimental import pallas as pl
from jax.experimental.pallas import tpu as pltpu
from jax.experimental.pallas import tpu_sc as plsc

VOCAB = 1000000
EMBED_DIM = 64
BATCH = 16384
HIST_LEN = 50

CHUNK = 128
NCHUNK_TOTAL = BATCH * HIST_LEN // CHUNK
NUM_WORKERS = 32
NCHUNK = NCHUNK_TOTAL // NUM_WORKERS
BTILES = BATCH // CHUNK
FTILES = EMBED_DIM // 8
NBUF = 5
LANES = 16


def _encode_kernel(table, idx_hbm, out, idx_v, rows, planes, gsems, osems):
    wid = lax.axis_index("s") * 2 + lax.axis_index("c")
    pltpu.sync_copy(idx_hbm.at[wid], idx_v)

    iota = lax.iota(jnp.int32, LANES)
    pks = [lax.bitwise_and(iota + k, LANES - 1) for k in range(LANES)]

    def start_gather(b, g):
        pltpu.async_copy(table.at[idx_v.at[g]], rows[b], gsems[b])

    def out_blocks(b, g):
        c = wid * NCHUNK + g
        t = c // BTILES
        bt = c - t * BTILES
        return [(planes[b].at[pl.ds(ft * 8, 8)], out.at[t, ft, bt])
                for ft in range(FTILES)]

    def start_out(b, g):
        for src, dst in out_blocks(b, g):
            pltpu.async_copy(src, dst, osems[b])

    def wait_out(b, g):
        for src, dst in out_blocks(b, g):
            pltpu.make_async_copy(src, dst, osems[b]).wait()

    def build_plane(b):
        @pl.loop(0, CHUNK, step=LANES)
        def _per_jblock(j0):
            row = iota + j0
            for d0 in range(0, EMBED_DIM, LANES):
                for k in range(LANES):
                    dvec = pks[k] + d0
                    v = plsc.load_gather(rows[b], [row, dvec])
                    plsc.store_scatter(planes[b], [dvec, row], v)

    for b in range(NBUF):
        start_gather(b, b)

    @pl.loop(0, NCHUNK, step=NBUF)
    def _body(g0):
        for b in range(NBUF):
            g = g0 + b

            @pl.when(g0 > 0)
            def _drain_prev_out():
                wait_out(b, g - NBUF)

            pltpu.make_async_copy(table.at[idx_v.at[g]], rows[b],
                                  gsems[b]).wait()
            build_plane(b)
            start_out(b, g)

            @pl.when(g + NBUF < NCHUNK)
            def _refill():
                start_gather(b, g + NBUF)

    for b in range(NBUF):
        wait_out(b, NCHUNK - NBUF + b)


def kernel(item_embeddings, batch_data):
    idx = batch_data.astype(jnp.int32).T.reshape(NUM_WORKERS, NCHUNK, CHUNK)
    mesh = plsc.VectorSubcoreMesh(core_axis_name="c", subcore_axis_name="s")
    tiles = pl.kernel(
        _encode_kernel,
        out_type=jax.ShapeDtypeStruct((HIST_LEN, FTILES, BTILES, 8, CHUNK),
                                      jnp.float32),
        mesh=mesh,
        scratch_types=[
            pltpu.VMEM((NCHUNK, CHUNK), jnp.int32),
            tuple(pltpu.VMEM((CHUNK, EMBED_DIM), jnp.float32)
                  for _ in range(NBUF)),
            tuple(pltpu.VMEM((EMBED_DIM, CHUNK), jnp.float32)
                  for _ in range(NBUF)),
            tuple(pltpu.SemaphoreType.DMA for _ in range(NBUF)),
            tuple(pltpu.SemaphoreType.DMA for _ in range(NBUF)),
        ],
        compiler_params=pltpu.CompilerParams(use_tc_tiling_on_sc=False,
                                             needs_layout_passes=False),
    )(item_embeddings, idx)
    return tiles.transpose(2, 4, 0, 1, 3).reshape(BATCH, HIST_LEN, EMBED_DIM)


# speedup vs baseline: 1.0116x; 1.0116x over previous
import functools

import jax
import jax.numpy as jnp
from jax import lax
from jax.experimental import pallas as pl
from jax.experimental.pallas import tpu as pltpu
from jax.experimental.pallas import tpu_sc as plsc

VOCAB = 1000000
EMBED_DIM = 64
BATCH = 16384
HIST_LEN = 50

CHUNK = 128
NCHUNK_TOTAL = BATCH * HIST_LEN // CHUNK
NUM_WORKERS = 32
NCHUNK = NCHUNK_TOTAL // NUM_WORKERS
BTILES = BATCH // CHUNK
FTILES = EMBED_DIM // 8
NBUF = 5
LANES = 16


def _encode_kernel(table, idx_hbm, out, idx_v, rows, planes, gsems, osems):
    wid = lax.axis_index("s") * 2 + lax.axis_index("c")
    pltpu.sync_copy(idx_hbm.at[wid], idx_v)

    iota = lax.iota(jnp.int32, LANES)
    pks = [lax.bitwise_and(iota + k, LANES - 1) for k in range(LANES)]

    def start_gather(b, g):
        pltpu.async_copy(table.at[idx_v.at[g]], rows[b], gsems[b])

    def out_block(g):
        c = wid * NCHUNK + g
        t = c // BTILES
        bt = c - t * BTILES
        return out.at[t, :, bt]

    def start_out(b, g):
        pltpu.async_copy(planes[b], out_block(g), osems[b])

    def wait_out(b, g):
        pltpu.make_async_copy(planes[b], out_block(g), osems[b]).wait()

    def build_plane(b):
        @pl.loop(0, CHUNK, step=LANES)
        def _per_jblock(j0):
            row = iota + j0
            for d0 in range(0, EMBED_DIM, LANES):
                for k in range(LANES):
                    dvec = pks[k] + d0
                    v = plsc.load_gather(rows[b], [row, dvec])
                    plsc.store_scatter(
                        planes[b],
                        [lax.shift_right_logical(dvec, 3),
                         lax.bitwise_and(dvec, 7), row], v)

    for b in range(NBUF):
        start_gather(b, b)

    @pl.loop(0, NCHUNK, step=NBUF)
    def _body(g0):
        for b in range(NBUF):
            g = g0 + b

            @pl.when(g0 > 0)
            def _drain_prev_out():
                wait_out(b, g - NBUF)

            pltpu.make_async_copy(table.at[idx_v.at[g]], rows[b],
                                  gsems[b]).wait()
            build_plane(b)
            start_out(b, g)

            @pl.when(g + NBUF < NCHUNK)
            def _refill():
                start_gather(b, g + NBUF)

    for b in range(NBUF):
        wait_out(b, NCHUNK - NBUF + b)


def kernel(item_embeddings, batch_data):
    idx = batch_data.astype(jnp.int32).T.reshape(NUM_WORKERS, NCHUNK, CHUNK)
    mesh = plsc.VectorSubcoreMesh(core_axis_name="c", subcore_axis_name="s")
    tiles = pl.kernel(
        _encode_kernel,
        out_type=jax.ShapeDtypeStruct((HIST_LEN, FTILES, BTILES, 8, CHUNK),
                                      jnp.float32),
        mesh=mesh,
        scratch_types=[
            pltpu.VMEM((NCHUNK, CHUNK), jnp.int32),
            tuple(pltpu.VMEM((CHUNK, EMBED_DIM), jnp.float32)
                  for _ in range(NBUF)),
            tuple(pltpu.VMEM((FTILES, 8, CHUNK), jnp.float32)
                  for _ in range(NBUF)),
            tuple(pltpu.SemaphoreType.DMA for _ in range(NBUF)),
            tuple(pltpu.SemaphoreType.DMA for _ in range(NBUF)),
        ],
        compiler_params=pltpu.CompilerParams(use_tc_tiling_on_sc=False,
                                             needs_layout_passes=False),
    )(item_embeddings, idx)
    return tiles.transpose(2, 4, 0, 1, 3).reshape(BATCH, HIST_LEN, EMBED_DIM)


# speedup vs baseline: 1.1976x; 1.1839x over previous
import jax
import jax.numpy as jnp
from jax import lax
from jax.experimental import pallas as pl
from jax.experimental.pallas import tpu as pltpu
from jax.experimental.pallas import tpu_sc as plsc

VOCAB = 1000000
EMBED_DIM = 64
BATCH = 16384
HIST_LEN = 50

CHUNK = 128
NCHUNK_TOTAL = BATCH * HIST_LEN // CHUNK
NUM_WORKERS = 32
NCHUNK = NCHUNK_TOTAL // NUM_WORKERS
BTILES = BATCH // CHUNK
FTILES = EMBED_DIM // 8
NBUF = 5
LANES = 16


def _encode_kernel(table, idx_hbm, out, idx_v, rows, planes, gsems, osems):
    wid = lax.axis_index("s") * 2 + lax.axis_index("c")
    pltpu.sync_copy(idx_hbm.at[wid], idx_v)

    iota = lax.iota(jnp.int32, LANES)
    pks = [lax.bitwise_and(iota + k, LANES - 1) for k in range(LANES)]

    def start_gather(b, g):
        pltpu.async_copy(table.at[idx_v.at[g]], rows[b], gsems[b])

    def out_block(g):
        c = wid * NCHUNK + g
        t = c // BTILES
        bt = c - t * BTILES
        return out.at[t, :, bt]

    def start_out(b, g):
        pltpu.async_copy(planes[b], out_block(g), osems[b])

    def wait_out(b, g):
        pltpu.make_async_copy(planes[b], out_block(g), osems[b]).wait()

    def build_plane(b):
        @plsc.parallel_loop(0, CHUNK, LANES, unroll=2)
        def _per_jblock(j0):
            row = iota + j0
            for d0 in range(0, EMBED_DIM, LANES):
                for k in range(LANES):
                    dvec = pks[k] + d0
                    v = plsc.load_gather(rows[b], [row, dvec])
                    plsc.store_scatter(
                        planes[b],
                        [lax.shift_right_logical(dvec, 3),
                         lax.bitwise_and(dvec, 7), row], v)

    for b in range(NBUF):
        start_gather(b, b)

    @pl.loop(0, NCHUNK, step=NBUF)
    def _body(g0):
        for b in range(NBUF):
            g = g0 + b

            @pl.when(g0 > 0)
            def _drain_prev_out():
                wait_out(b, g - NBUF)

            pltpu.make_async_copy(table.at[idx_v.at[g]], rows[b],
                                  gsems[b]).wait()
            build_plane(b)
            start_out(b, g)

            @pl.when(g + NBUF < NCHUNK)
            def _refill():
                start_gather(b, g + NBUF)

    for b in range(NBUF):
        wait_out(b, NCHUNK - NBUF + b)


def kernel(item_embeddings, batch_data):
    idx = batch_data.astype(jnp.int32).T.reshape(NUM_WORKERS, NCHUNK, CHUNK)
    mesh = plsc.VectorSubcoreMesh(core_axis_name="c", subcore_axis_name="s")
    tiles = pl.kernel(
        _encode_kernel,
        out_type=jax.ShapeDtypeStruct((HIST_LEN, FTILES, BTILES, 8, CHUNK),
                                      jnp.float32),
        mesh=mesh,
        scratch_types=[
            pltpu.VMEM((NCHUNK, CHUNK), jnp.int32),
            tuple(pltpu.VMEM((CHUNK, EMBED_DIM), jnp.float32)
                  for _ in range(NBUF)),
            tuple(pltpu.VMEM((FTILES, 8, CHUNK), jnp.float32)
                  for _ in range(NBUF)),
            tuple(pltpu.SemaphoreType.DMA for _ in range(NBUF)),
            tuple(pltpu.SemaphoreType.DMA for _ in range(NBUF)),
        ],
        compiler_params=pltpu.CompilerParams(use_tc_tiling_on_sc=False,
                                             needs_layout_passes=False),
    )(item_embeddings, idx)
    return tiles.transpose(2, 4, 0, 1, 3).reshape(BATCH, HIST_LEN, EMBED_DIM)


# speedup vs baseline: 1.3741x; 1.1474x over previous
import jax
import jax.numpy as jnp
from jax import lax
from jax.experimental import pallas as pl
from jax.experimental.pallas import tpu as pltpu
from jax.experimental.pallas import tpu_sc as plsc

VOCAB = 1000000
EMBED_DIM = 64
BATCH = 16384
HIST_LEN = 50

CHUNK = 128
NCHUNK_TOTAL = BATCH * HIST_LEN // CHUNK
NUM_WORKERS = 32
NCHUNK = NCHUNK_TOTAL // NUM_WORKERS
BTILES = BATCH // CHUNK
FTILES = EMBED_DIM // 8
NBUF = 5
LANES = 16


def _encode_kernel(table, idx_hbm, out, idx_v, rows, planes, gsems, osems):
    wid = lax.axis_index("s") * 2 + lax.axis_index("c")
    pltpu.sync_copy(idx_hbm.at[wid], idx_v)

    iota = lax.iota(jnp.int32, LANES)
    pks = [lax.bitwise_and(iota + k, LANES - 1) for k in range(LANES)]

    def start_gather(b, g):
        pltpu.async_copy(table.at[idx_v.at[g]], rows[b], gsems[b])

    def out_block(g):
        c = wid * NCHUNK + g
        t = c // BTILES
        bt = c - t * BTILES
        return out.at[t, :, bt]

    def start_out(b, g):
        pltpu.async_copy(planes[b], out_block(g), osems[b])

    def wait_out(b, g):
        pltpu.make_async_copy(planes[b], out_block(g), osems[b]).wait()

    def build_plane(b):
        @plsc.parallel_loop(0, CHUNK, LANES, unroll=4)
        def _per_jblock(j0):
            row = iota + j0
            for d0 in range(0, EMBED_DIM, LANES):
                for k in range(LANES):
                    dvec = pks[k] + d0
                    v = plsc.load_gather(rows[b], [row, dvec])
                    plsc.store_scatter(
                        planes[b],
                        [lax.shift_right_logical(dvec, 3),
                         lax.bitwise_and(dvec, 7), row], v)

    for b in range(NBUF):
        start_gather(b, b)

    @pl.loop(0, NCHUNK, step=NBUF)
    def _body(g0):
        for b in range(NBUF):
            g = g0 + b

            @pl.when(g0 > 0)
            def _drain_prev_out():
                wait_out(b, g - NBUF)

            pltpu.make_async_copy(table.at[idx_v.at[g]], rows[b],
                                  gsems[b]).wait()
            build_plane(b)
            start_out(b, g)

            @pl.when(g + NBUF < NCHUNK)
            def _refill():
                start_gather(b, g + NBUF)

    for b in range(NBUF):
        wait_out(b, NCHUNK - NBUF + b)


def kernel(item_embeddings, batch_data):
    idx = batch_data.astype(jnp.int32).T.reshape(NUM_WORKERS, NCHUNK, CHUNK)
    mesh = plsc.VectorSubcoreMesh(core_axis_name="c", subcore_axis_name="s")
    tiles = pl.kernel(
        _encode_kernel,
        out_type=jax.ShapeDtypeStruct((HIST_LEN, FTILES, BTILES, 8, CHUNK),
                                      jnp.float32),
        mesh=mesh,
        scratch_types=[
            pltpu.VMEM((NCHUNK, CHUNK), jnp.int32),
            tuple(pltpu.VMEM((CHUNK, EMBED_DIM), jnp.float32)
                  for _ in range(NBUF)),
            tuple(pltpu.VMEM((FTILES, 8, CHUNK), jnp.float32)
                  for _ in range(NBUF)),
            tuple(pltpu.SemaphoreType.DMA for _ in range(NBUF)),
            tuple(pltpu.SemaphoreType.DMA for _ in range(NBUF)),
        ],
        compiler_params=pltpu.CompilerParams(use_tc_tiling_on_sc=False,
                                             needs_layout_passes=False),
    )(item_embeddings, idx)
    return tiles.transpose(2, 4, 0, 1, 3).reshape(BATCH, HIST_LEN, EMBED_DIM)
